# baseline (speedup 1.0000x reference)
import functools

import jax
import jax.numpy as jnp
from jax import lax
from jax.experimental import pallas as pl
from jax.experimental.pallas import tpu as pltpu
from jax.experimental.pallas import tpu_sc as plsc

VOCAB = 1000000
DIM = 32
BATCH = 16384
HIST = 50
TOTAL = BATCH * HIST

_info = plsc.get_sparse_core_info()
_NC = _info.num_cores
_NS = _info.num_subcores
_NW = _NC * _NS
_BW = BATCH // _NW
_BT = _BW // 128
_NG = 2
_NO = 2


def _make_kernel():
    mesh = plsc.VectorSubcoreMesh(core_axis_name="c", subcore_axis_name="s")

    @functools.partial(
        pl.kernel,
        mesh=mesh,
        out_type=jax.ShapeDtypeStruct((HIST, DIM // 8, BATCH // 128, 8, 128),
                                      jnp.float32),
        scratch_types=[
            pltpu.VMEM((HIST * _BW,), jnp.int32),
            [pltpu.VMEM((_BW, DIM), jnp.float32) for _ in range(_NG)],
            [pltpu.VMEM((DIM // 8, _BT, 8, 128), jnp.float32) for _ in range(_NO)],
            pltpu.SemaphoreType.DMA,
            [pltpu.SemaphoreType.DMA for _ in range(_NG)],
            [pltpu.SemaphoreType.DMA for _ in range(_NO)],
        ],
        compiler_params=pltpu.CompilerParams(
            use_tc_tiling_on_sc=False, needs_layout_passes=False
        ),
    )
    def k(idx_hbm, table_hbm, out_hbm, idx_all, rows, obuf, isem, gsems, ssems):
        w = lax.axis_index("s") * _NC + lax.axis_index("c")
        b0 = w * _BW

        idx_copies = []
        for h in range(HIST):
            idx_copies.append(pltpu.async_copy(
                idx_hbm.at[pl.ds(h * BATCH + b0, _BW)],
                idx_all.at[pl.ds(h * _BW, _BW)],
                isem,
            ))
        for c in idx_copies:
            c.wait()

        def start_gather(h, p):
            return pltpu.async_copy(
                table_hbm.at[idx_all.at[pl.ds(h * _BW, _BW)]],
                rows[p],
                gsems[p],
            )

        bvecs = [lax.iota(jnp.int32, 16) + 16 * j for j in range(_BW // 16)]

        def transpose_unit(p, sp):
            rp, op = rows[p], obuf[sp]

            def dtbody(dt, _):
                for di in range(8):
                    d = dt * 8 + di
                    dvec = jnp.full((16,), d, jnp.int32)
                    vs = [plsc.load_gather(rp, [bvecs[j], dvec])
                          for j in range(_BW // 16)]
                    for j in range(_BW // 16):
                        op[dt, j // 8, di, pl.ds((j % 8) * 16, 16)] = vs[j]
                return _

            lax.fori_loop(0, DIM // 8, dtbody, None)

        for h0 in range(_NG):
            start_gather(h0, h0)

        def store_ref(h):
            return out_hbm.at[h, :, pl.ds(_BT * w, _BT)]

        def unit(h, p, sp):
            pltpu.make_async_copy(
                table_hbm.at[idx_all.at[pl.ds(h * _BW, _BW)]],
                rows[p],
                gsems[p],
            ).wait()

            @pl.when(h >= _NO)
            def _():
                pltpu.make_async_copy(obuf[sp], store_ref(h), ssems[sp]).wait()

            transpose_unit(p, sp)

            @pl.when(h + _NG < HIST)
            def _():
                start_gather(h + _NG, p)

            pltpu.async_copy(obuf[sp], store_ref(h), ssems[sp])

        def base_body(base, _):
            for q in range(2):
                unit(2 * base + q, q % _NG, q % _NO)
            return _

        lax.fori_loop(0, HIST // 2, base_body, None)
        for h in range(HIST - _NO, HIST):
            pltpu.make_async_copy(obuf[h % _NO], store_ref(h), ssems[h % _NO]).wait()

    return k


_k = _make_kernel()


@jax.jit
def kernel(x, embeddings):
    tpad = jnp.pad(embeddings, ((0, 0), (0, 128 - DIM))).reshape(4 * VOCAB, DIM)
    idx = x.T.reshape(TOTAL).astype(jnp.int32) * 4
    y6 = _k(idx, tpad)
    return y6.transpose(2, 4, 0, 1, 3).reshape(BATCH, HIST, DIM)
